# baseline (speedup 1.0000x reference)
import functools

import jax
import jax.numpy as jnp
from jax import lax
from jax.experimental import pallas as pl
from jax.experimental.pallas import tpu as pltpu
from jax.experimental.pallas import tpu_sc as plsc

C = 128
K = 80
N_POINTS = 100000
N_CHUNKS = N_POINTS // K
NW = 32
OUTER = (N_CHUNKS + NW - 1) // NW
JGROUPS = C // 16


def _sc_loss(imgt, idx, feature):
    mesh = plsc.VectorSubcoreMesh(core_axis_name="c", subcore_axis_name="s")

    @functools.partial(
        pl.kernel,
        mesh=mesh,
        out_type=jax.ShapeDtypeStruct((NW, JGROUPS, 16), jnp.float32),
        scratch_types=[
            pltpu.VMEM((K,), jnp.int32),
            pltpu.VMEM((K, C), jnp.float32),
            pltpu.VMEM((K, C), jnp.float32),
            pltpu.VMEM((JGROUPS, 16), jnp.float32),
            pltpu.SemaphoreType.DMA,
        ],
    )
    def k(imgt_hbm, idx_hbm, feat_hbm, out_hbm, idx_v, img_v, feat_v, acc_v,
          sem):
        wid = lax.axis_index("s") * 2 + lax.axis_index("c")

        for j in range(JGROUPS):
            acc_v[j, :] = jnp.zeros((16,), jnp.float32)

        def chunk_body(ci, _):
            ch = wid + ci * NW

            @pl.when(ch < N_CHUNKS)
            def _():
                base = ch * K
                pltpu.sync_copy(idx_hbm.at[pl.ds(base, K)], idx_v)
                cp_g = pltpu.async_copy(imgt_hbm.at[idx_v], img_v, sem)
                cp_f = pltpu.async_copy(feat_hbm.at[pl.ds(base, K)], feat_v,
                                        sem)
                cp_g.wait()
                cp_f.wait()

                def row_body(r, accs):
                    new = []
                    for j in range(JGROUPS):
                        g = img_v[r, pl.ds(j * 16, 16)]
                        t = feat_v[r, pl.ds(j * 16, 16)]
                        d = g - t * (1.0 / 200.0)
                        new.append(accs[j] + d * d)
                    return tuple(new)

                accs = lax.fori_loop(
                    0, K, row_body,
                    tuple(acc_v[j, :] for j in range(JGROUPS)))
                for j in range(JGROUPS):
                    acc_v[j, :] = accs[j]

            return 0

        lax.fori_loop(0, OUTER, chunk_body, 0)
        pltpu.sync_copy(acc_v, out_hbm.at[wid])

    return k(imgt, idx, feature)


def kernel(image, points, feature):
    imgt = image[0].reshape(C, -1).T
    idx = points[:, 0] * 4096 + points[:, 1] * 64 + points[:, 2]
    partials = _sc_loss(imgt, idx.astype(jnp.int32), feature)
    return jnp.sum(partials)


# speedup vs baseline: 1.8049x; 1.8049x over previous
import functools

import jax
import jax.numpy as jnp
from jax import lax
from jax.experimental import pallas as pl
from jax.experimental.pallas import tpu as pltpu
from jax.experimental.pallas import tpu_sc as plsc

C = 128
K = 80
N_POINTS = 100000
N_CHUNKS = N_POINTS // K
NW = 32
MAXCH = 40
IDX_ROWS = MAXCH * NW
JGROUPS = C // 16


def _sc_loss(imgt, idx2d, feature):
    mesh = plsc.VectorSubcoreMesh(core_axis_name="c", subcore_axis_name="s")

    @functools.partial(
        pl.kernel,
        mesh=mesh,
        out_type=jax.ShapeDtypeStruct((NW, JGROUPS, 16), jnp.float32),
        scratch_types=[
            pltpu.VMEM((MAXCH, K), jnp.int32),
            pltpu.VMEM((K, C), jnp.float32),
            pltpu.VMEM((K, C), jnp.float32),
            pltpu.VMEM((K, C), jnp.float32),
            pltpu.VMEM((K, C), jnp.float32),
            pltpu.VMEM((JGROUPS, 16), jnp.float32),
            pltpu.SemaphoreType.DMA,
            pltpu.SemaphoreType.DMA,
            pltpu.SemaphoreType.DMA,
            pltpu.SemaphoreType.DMA,
        ],
    )
    def k(imgt_hbm, idx_hbm, feat_hbm, out_hbm, idx_v, img0_v, img1_v,
          feat0_v, feat1_v, acc_v, sg0, sg1, sf0, sf1):
        wid = lax.axis_index("s") * 2 + lax.axis_index("c")
        base_ch = MAXCH * wid
        nch = jnp.minimum(MAXCH, jnp.maximum(N_CHUNKS - base_ch, 0))

        pltpu.sync_copy(idx_hbm.at[pl.ds(base_ch, MAXCH)], idx_v)

        for j in range(JGROUPS):
            acc_v[j, :] = jnp.zeros((16,), jnp.float32)

        bufs = ((img0_v, feat0_v, sg0, sf0), (img1_v, feat1_v, sg1, sf1))

        def issue(ci, b):
            img_b, feat_b, sg, sf = bufs[b]

            @pl.when(ci < nch)
            def _():
                pltpu.async_copy(imgt_hbm.at[idx_v.at[ci]], img_b, sg)
                pltpu.async_copy(feat_hbm.at[pl.ds((base_ch + ci) * K, K)],
                                 feat_b, sf)

        def consume(ci, b):
            img_b, feat_b, sg, sf = bufs[b]

            @pl.when(ci < nch)
            def _():
                pltpu.make_async_copy(imgt_hbm.at[idx_v.at[ci]], img_b,
                                      sg).wait()
                pltpu.make_async_copy(
                    feat_hbm.at[pl.ds((base_ch + ci) * K, K)], feat_b,
                    sf).wait()

                def row_body(r, accs):
                    new = list(accs)
                    for u in range(2):
                        for j in range(JGROUPS):
                            g = img_b[2 * r + u, pl.ds(j * 16, 16)]
                            t = feat_b[2 * r + u, pl.ds(j * 16, 16)]
                            d = g * 200.0 - t
                            new[j] = new[j] + d * d
                    return tuple(new)

                accs = lax.fori_loop(
                    0, K // 2, row_body,
                    tuple(acc_v[j, :] for j in range(JGROUPS)))
                for j in range(JGROUPS):
                    acc_v[j, :] = accs[j]

        issue(0, 0)
        issue(1, 1)

        def outer(ci, _):
            consume(ci, 0)
            issue(ci + 2, 0)
            consume(ci + 1, 1)
            issue(ci + 3, 1)
            return 0

        lax.fori_loop(0, MAXCH // 2, lambda i, c: outer(2 * i, c), 0)

        for j in range(JGROUPS):
            acc_v[j, :] = acc_v[j, :] * (1.0 / 40000.0)
        pltpu.sync_copy(acc_v, out_hbm.at[wid])

    return k(imgt, idx2d, feature)


def kernel(image, points, feature):
    imgt = image[0].reshape(C, -1).T
    idx = points[:, 0] * 4096 + points[:, 1] * 64 + points[:, 2]
    idx2d = jnp.zeros((IDX_ROWS * K,), jnp.int32).at[:N_POINTS].set(
        idx.astype(jnp.int32)).reshape(IDX_ROWS, K)
    partials = _sc_loss(imgt, idx2d, feature)
    return jnp.sum(partials)
